# baseline (speedup 1.0000x reference)
import functools

import jax
import jax.numpy as jnp
from jax import lax
from jax.experimental import pallas as pl
from jax.experimental.pallas import tpu as pltpu
from jax.experimental.pallas import tpu_sc as plsc


def kernel(table, seq_len):
    n, d = table.shape
    info = plsc.get_sparse_core_info()
    num_cores, num_subcores, lanes = (
        info.num_cores, info.num_subcores, info.num_lanes)
    num_workers = num_cores * num_subcores
    rows_per_w = n // num_workers

    mesh = plsc.VectorSubcoreMesh(core_axis_name="c", subcore_axis_name="s")

    @functools.partial(
        pl.kernel,
        mesh=mesh,
        out_type=jax.ShapeDtypeStruct((n, d), table.dtype),
        scratch_types=[
            pltpu.SemaphoreType.DMA,
        ],
    )
    def cp(table_hbm, out_hbm, csem):
        wid = lax.axis_index("s") * num_cores + lax.axis_index("c")
        base = wid * rows_per_w
        pltpu.async_copy(
            table_hbm.at[pl.ds(base, rows_per_w)],
            out_hbm.at[pl.ds(base, rows_per_w)],
            csem,
        ).wait()

    return cp(table)


# speedup vs baseline: 6.0782x; 6.0782x over previous
import functools

import jax
import jax.numpy as jnp
from jax import lax
from jax.experimental import pallas as pl
from jax.experimental.pallas import tpu as pltpu
from jax.experimental.pallas import tpu_sc as plsc

_CHUNK = 32


def kernel(table, seq_len):
    n, d = table.shape
    info = plsc.get_sparse_core_info()
    num_cores, num_subcores, lanes = (
        info.num_cores, info.num_subcores, info.num_lanes)
    num_workers = num_cores * num_subcores
    rows_per_w = n // num_workers
    n_chunks = rows_per_w // _CHUNK

    seq_len_arr = jnp.broadcast_to(jnp.asarray(seq_len, jnp.int32) - 1, (16,))

    mesh = plsc.VectorSubcoreMesh(core_axis_name="c", subcore_axis_name="s")

    @functools.partial(
        pl.kernel,
        mesh=mesh,
        out_type=jax.ShapeDtypeStruct((n, d), table.dtype),
        scratch_types=[
            pltpu.VMEM((n_chunks, _CHUNK), jnp.int32),
            pltpu.VMEM((rows_per_w, d), jnp.float32),
            pltpu.VMEM((16,), jnp.int32),
            pltpu.SemaphoreType.DMA,
            pltpu.SemaphoreType.DMA,
        ],
    )
    def emb(table_hbm, slen_hbm, out_hbm, idx_v, rows_v, slen_v, gsem, wsem):
        wid = lax.axis_index("s") * num_cores + lax.axis_index("c")
        base = wid * rows_per_w
        pltpu.sync_copy(slen_hbm, slen_v)
        clamp = slen_v[...]
        gathers = []
        for j in range(n_chunks):
            for i in range(_CHUNK // lanes):
                rows = base + (j * _CHUNK + i * lanes) + lax.iota(jnp.int32, lanes)
                idx_v[j, pl.ds(i * lanes, lanes)] = jnp.minimum(rows, clamp)
            gathers.append(pltpu.async_copy(
                table_hbm.at[idx_v.at[j]],
                rows_v.at[pl.ds(j * _CHUNK, _CHUNK)],
                gsem,
            ))
        writes = []
        for j in range(n_chunks):
            gathers[j].wait()
            writes.append(pltpu.async_copy(
                rows_v.at[pl.ds(j * _CHUNK, _CHUNK)],
                out_hbm.at[pl.ds(base + j * _CHUNK, _CHUNK)],
                wsem,
            ))
        for w in writes:
            w.wait()

    return emb(table, seq_len_arr)
